# baseline (speedup 1.0000x reference)
import jax
import jax.numpy as jnp
from jax import lax
from jax.experimental import pallas as pl
from jax.experimental.pallas import tpu as pltpu
from jax.experimental.pallas import tpu_sc as plsc

N = 50000
K = 32
KP = K + 1
L = 16
NW = 32
GROUPS_PER_TILE = 98
PER_W = GROUPS_PER_TILE * L
NCHUNK = 14
GROUPS_PER_CHUNK = GROUPS_PER_TILE // NCHUNK
CHUNK_NODES = GROUPS_PER_CHUNK * L

MEAN_PER_SUB = 196


def _moran_body(x_hbm, w_hbm, ids_hbm, out_hbm,
                x_v, ids_a, ids_b, wts_a, wts_b, ids_p, wts_p,
                out_v, red_v, shared_red,
                sem_x, sem_ids, sem_wts):
    cid = lax.axis_index("c")
    sid = lax.axis_index("s")
    wid = sid * 2 + cid
    base = jnp.where(wid == NW - 1, N - PER_W, wid * PER_W)

    ids_bufs = (ids_a, ids_b)
    wts_bufs = (wts_a, wts_b)

    def issue(ci):
        off = base + ci * CHUNK_NODES
        h1 = pltpu.async_copy(ids_hbm.at[pl.ds(off, CHUNK_NODES)],
                              ids_bufs[ci % 2], sem_ids)
        h2 = pltpu.async_copy(w_hbm.at[pl.ds(off, CHUNK_NODES)],
                              wts_bufs[ci % 2], sem_wts)
        return (h1, h2)

    cp_x = pltpu.async_copy(x_hbm, x_v, sem_x)
    pending = {0: issue(0), 1: issue(1)}
    cp_x.wait()

    mstart = sid * MEAN_PER_SUB * L
    def mean_body(i, accs):
        b = mstart + i * (4 * L)
        a0, a1, a2, a3 = accs
        a0 = a0 + x_v[pl.ds(b, L)]
        a1 = a1 + x_v[pl.ds(b + L, L)]
        a2 = a2 + x_v[pl.ds(b + 2 * L, L)]
        a3 = a3 + x_v[pl.ds(b + 3 * L, L)]
        return (a0, a1, a2, a3)
    z = jnp.zeros((L,), jnp.float32)
    nquad = jnp.where(sid == 15, 45, 49)
    accs = lax.fori_loop(0, nquad, mean_body, (z, z, z, z))
    part = accs[0] + accs[1] + accs[2] + accs[3]

    def mean_tail(i, p):
        return p + x_v[pl.ds(mstart + (180 + i) * L, L)]
    part = jnp.where(sid == 15, lax.fori_loop(0, 5, mean_tail, z), z) + part

    red_v[pl.ds(0, L)] = part
    pltpu.sync_copy(red_v.at[pl.ds(0, L)], shared_red.at[pl.ds(sid * L, L)])
    plsc.subcore_barrier()
    pltpu.sync_copy(shared_red, red_v)
    tot = z
    for r in range(L):
        tot = tot + red_v[pl.ds(r * L, L)]
    s = tot[0]
    for i in range(1, L):
        s = s + tot[i]
    m = s * (1.0 / N)

    iota = lax.iota(jnp.int32, L)
    iota_kp = iota * KP

    for ci in range(NCHUNK):
        ib = ids_bufs[ci % 2]
        wb = wts_bufs[ci % 2]
        h1, h2 = pending.pop(ci)
        h1.wait()
        h2.wait()

        def repack(n, _, ib=ib, wb=wb):
            d0 = n * KP
            ids_p[pl.ds(d0, L)] = ib[n, pl.ds(0, L)]
            ids_p[pl.ds(d0 + L, L)] = ib[n, pl.ds(L, L)]
            wts_p[pl.ds(d0, L)] = wb[n, pl.ds(0, L)]
            wts_p[pl.ds(d0 + L, L)] = wb[n, pl.ds(L, L)]
            return 0
        lax.fori_loop(0, CHUNK_NODES, repack, 0)

        def grp(g, _, ci=ci):
            idx_base = g * (L * KP) + iota_kp
            z16 = jnp.zeros((L,), jnp.float32)
            acc = [[z16, z16, z16], [z16, z16, z16]]
            for j in range(K):
                idx = idx_base + j
                nid = plsc.load_gather(ids_p, [idx])
                w = plsc.load_gather(wts_p, [idx])
                xg = plsc.load_gather(x_v, [nid])
                t = w * xg
                a = acc[j % 2]
                a[0] = a[0] + w
                a[1] = a[1] + t
                a[2] = a[2] + t * xg
            sw = acc[0][0] + acc[1][0]
            swx = acc[0][1] + acc[1][1]
            swxx = acc[0][2] + acc[1][2]
            goff = (ci * GROUPS_PER_CHUNK + g) * L
            own = x_v[pl.ds(base + goff, L)]
            xa = own - m
            num = swx - m * sw
            den = swxx - m * (2.0 * swx - m * sw)
            out_v[pl.ds(goff, L)] = xa * num * (K - 1.0) / den
            return 0

        lax.fori_loop(0, GROUPS_PER_CHUNK, grp, 0)
        if ci + 2 < NCHUNK:
            pending[ci + 2] = issue(ci + 2)

    pltpu.sync_copy(out_v, out_hbm.at[pl.ds(base, PER_W)])


@jax.jit
def _moran_sc(x, wts, ids):
    mesh = plsc.VectorSubcoreMesh(core_axis_name="c", subcore_axis_name="s")
    return pl.kernel(
        _moran_body,
        out_type=jax.ShapeDtypeStruct((N,), jnp.float32),
        mesh=mesh,
        compiler_params=pltpu.CompilerParams(needs_layout_passes=False),
        scratch_types=[
            pltpu.VMEM((N,), jnp.float32),
            pltpu.VMEM((CHUNK_NODES, K), jnp.int32),
            pltpu.VMEM((CHUNK_NODES, K), jnp.int32),
            pltpu.VMEM((CHUNK_NODES, K), jnp.float32),
            pltpu.VMEM((CHUNK_NODES, K), jnp.float32),
            pltpu.VMEM((CHUNK_NODES * KP,), jnp.int32),
            pltpu.VMEM((CHUNK_NODES * KP,), jnp.float32),
            pltpu.VMEM((PER_W,), jnp.float32),
            pltpu.VMEM((16 * L,), jnp.float32),
            pltpu.VMEM_SHARED((16 * L,), jnp.float32),
            pltpu.SemaphoreType.DMA,
            pltpu.SemaphoreType.DMA,
            pltpu.SemaphoreType.DMA,
        ],
    )(x, wts, ids)


def kernel(X, neighbor_weights, neighbor_ids):
    return _moran_sc(X, neighbor_weights, neighbor_ids.astype(jnp.int32))
